# baseline (speedup 1.0000x reference)
import functools

import jax
import jax.numpy as jnp
from jax import lax
from jax.experimental import pallas as pl
from jax.experimental.pallas import tpu as pltpu
from jax.experimental.pallas import tpu_sc as plsc

E = 8
D = 768
H = 768
N = 8192
T = 256
NT = N // T + E
NPAD = NT * T
RT = 1024
RG = N // RT

NW = 32
CHUNK = 64
TPW = N // NW
CPW = TPW // CHUNK
DP = 896


def _router_body(x_ref, wt_ref, b_ref, eid_ref, prob_ref, counts_ref,
                 bloss_ref, cnt_s, ps_s):
    i = pl.program_id(0)

    @pl.when(i == 0)
    def _init():
        cnt_s[...] = jnp.zeros_like(cnt_s)
        ps_s[...] = jnp.zeros_like(ps_s)

    x = x_ref[...]
    logits = jnp.dot(x, wt_ref[...], preferred_element_type=jnp.float32)
    logits = logits + b_ref[...]
    m = jnp.max(logits, axis=1, keepdims=True)
    ex = jnp.exp(logits - m)
    s = jnp.sum(ex, axis=1, keepdims=True)
    probs = ex / s
    pmax = jnp.max(probs, axis=1, keepdims=True)
    iota_e = lax.broadcasted_iota(jnp.int32, (RT, E), 1)
    eid = jnp.min(jnp.where(probs == pmax, iota_e, E), axis=1, keepdims=True)
    eid_ref[...] = eid
    prob_ref[...] = pmax
    oh = (eid == iota_e).astype(jnp.float32)
    cnt_s[...] += jnp.sum(oh, axis=0, keepdims=True)
    ps_s[...] += jnp.sum(probs, axis=0, keepdims=True)
    counts_ref[...] = cnt_s[...]
    bloss_ref[...] = (E / (N * N)) * jnp.sum(
        cnt_s[...] * ps_s[...], axis=1, keepdims=True)


def _router(x2d, wt, rb):
    return pl.pallas_call(
        _router_body,
        grid=(RG,),
        in_specs=[
            pl.BlockSpec((RT, D), lambda i: (i, 0)),
            pl.BlockSpec((D, E), lambda i: (0, 0)),
            pl.BlockSpec((1, E), lambda i: (0, 0)),
        ],
        out_specs=[
            pl.BlockSpec((RT, 1), lambda i: (i, 0)),
            pl.BlockSpec((RT, 1), lambda i: (i, 0)),
            pl.BlockSpec((1, E), lambda i: (0, 0)),
            pl.BlockSpec((1, 1), lambda i: (0, 0)),
        ],
        out_shape=[
            jax.ShapeDtypeStruct((N, 1), jnp.int32),
            jax.ShapeDtypeStruct((N, 1), jnp.float32),
            jax.ShapeDtypeStruct((1, E), jnp.float32),
            jax.ShapeDtypeStruct((1, 1), jnp.float32),
        ],
        scratch_shapes=[
            pltpu.VMEM((1, E), jnp.float32),
            pltpu.VMEM((1, E), jnp.float32),
        ],
        compiler_params=pltpu.CompilerParams(
            dimension_semantics=("arbitrary",)),
    )(x2d, wt, rb)


def _meta_body(counts_ref, eid_ref, pdest_ref, eot_ref, run_s):
    i = pl.program_id(0)

    @pl.when(i == 0)
    def _init():
        run_s[...] = jnp.zeros_like(run_s)

    poff = []
    off = 0.0
    for e in range(E):
        c = counts_ref[0, e]
        pc = jnp.ceil(c / T) * T
        poff.append(off)
        off = off + pc

    iota_lane = lax.broadcasted_iota(jnp.int32, (1, E), 1)
    base = jnp.zeros((1, E), jnp.float32)
    for e in range(E):
        base = base + jnp.where(iota_lane == e, poff[e], 0.0)
    base = base + run_s[...]

    eid = eid_ref[...]
    iota_e = lax.broadcasted_iota(jnp.int32, (RT, E), 1)
    oh = (eid == iota_e).astype(jnp.float32)
    csum = oh
    k = 1
    while k < RT:
        csum = csum + jnp.concatenate(
            [jnp.zeros((k, E), jnp.float32), csum[:-k, :]], axis=0)
        k *= 2
    pd = jnp.sum(oh * (base + csum - 1.0), axis=1, keepdims=True)
    pdest_ref[...] = pd.astype(jnp.int32)
    run_s[...] += jnp.sum(oh, axis=0, keepdims=True)

    tgrid = lax.broadcasted_iota(jnp.int32, (1, 128), 1).astype(jnp.float32) * T
    acc = jnp.zeros((1, 128), jnp.float32)
    for e in range(E):
        acc = acc + (poff[e] <= tgrid).astype(jnp.float32)
    eot_ref[...] = (acc - 1.0).astype(jnp.int32)


def _meta(counts, eid):
    return pl.pallas_call(
        _meta_body,
        grid=(RG,),
        in_specs=[
            pl.BlockSpec(memory_space=pltpu.SMEM),
            pl.BlockSpec((RT, 1), lambda i: (i, 0)),
        ],
        out_specs=[
            pl.BlockSpec((RT, 1), lambda i: (i, 0)),
            pl.BlockSpec((1, 128), lambda i: (0, 0)),
        ],
        out_shape=[
            jax.ShapeDtypeStruct((N, 1), jnp.int32),
            jax.ShapeDtypeStruct((1, 128), jnp.int32),
        ],
        scratch_shapes=[pltpu.VMEM((1, E), jnp.float32)],
        compiler_params=pltpu.CompilerParams(
            dimension_semantics=("arbitrary",)),
    )(counts, eid)


@functools.cache
def _sc_kernels():
    mesh = plsc.VectorSubcoreMesh(core_axis_name="c", subcore_axis_name="s")

    @functools.partial(
        pl.kernel,
        out_type=jax.ShapeDtypeStruct((NPAD, DP), jnp.float32),
        mesh=mesh,
        scratch_types=[
            pltpu.VMEM((CPW, CHUNK), jnp.int32),
            pltpu.VMEM((2, CHUNK, DP), jnp.float32),
            pltpu.VMEM((TPW,), jnp.float32),
            pltpu.SemaphoreType.DMA,
            pltpu.SemaphoreType.DMA,
            pltpu.SemaphoreType.DMA,
        ],
        compiler_params=pltpu.CompilerParams(needs_layout_passes=False))
    def sc_scatter(x_hbm, pdest_hbm, prob_hbm, xpad_hbm,
                   idx_v, rows_v, prob_v, sem_i, sem_s0, sem_s1):
        wid = lax.axis_index("s") * 2 + lax.axis_index("c")
        wbase = wid * TPW
        iota16 = lax.broadcasted_iota(jnp.int32, (16,), 0)
        col = jnp.full((16,), D, jnp.int32)
        sems = [sem_s0, sem_s1]
        loads = [pltpu.make_async_copy(
            pdest_hbm.at[pl.ds(wbase + ch * CHUNK, CHUNK)], idx_v.at[ch],
            sem_i) for ch in range(CPW)]
        loads.append(pltpu.make_async_copy(
            prob_hbm.at[pl.ds(wbase, TPW)], prob_v, sem_i))
        for cp in loads:
            cp.start()
        for cp in loads:
            cp.wait()
        scatters = [None, None]
        for ch in range(CPW):
            slot = ch % 2
            if scatters[slot] is not None:
                scatters[slot].wait()
            pltpu.sync_copy(x_hbm.at[pl.ds(wbase + ch * CHUNK, CHUNK)],
                            rows_v.at[slot].at[:, pl.ds(0, D)])
            for g in range(CHUNK // 16):
                vals = prob_v[pl.ds(ch * CHUNK + 16 * g, 16)]
                plsc.store_scatter(rows_v.at[slot],
                                   [iota16 + 16 * g, col], vals)
            scatters[slot] = pltpu.make_async_copy(
                rows_v.at[slot], xpad_hbm.at[idx_v.at[ch]], sems[slot])
            scatters[slot].start()
        for cp in scatters:
            cp.wait()

    @functools.partial(
        pl.kernel,
        out_type=jax.ShapeDtypeStruct((N, D), jnp.float32),
        mesh=mesh,
        scratch_types=[
            pltpu.VMEM((TPW,), jnp.int32),
            pltpu.VMEM((2, CHUNK, D), jnp.float32),
            pltpu.SemaphoreType.DMA,
            pltpu.SemaphoreType.DMA,
        ])
    def sc_gather(opad_hbm, pdest_hbm, out_hbm, idx_v, rows_v, sem_g0, sem_g1):
        wid = lax.axis_index("s") * 2 + lax.axis_index("c")
        wbase = wid * TPW
        sems = [sem_g0, sem_g1]
        pltpu.sync_copy(pdest_hbm.at[pl.ds(wbase, TPW)], idx_v)

        def gather(ch):
            cp = pltpu.make_async_copy(
                opad_hbm.at[idx_v.at[pl.ds(ch * CHUNK, CHUNK)]],
                rows_v.at[ch % 2], sems[ch % 2])
            cp.start()
            return cp

        pending = gather(0)
        for ch in range(CPW):
            nxt = gather(ch + 1) if ch + 1 < CPW else None
            pending.wait()
            pltpu.sync_copy(rows_v.at[ch % 2],
                            out_hbm.at[pl.ds(wbase + ch * CHUNK, CHUNK)])
            pending = nxt

    return sc_scatter, sc_gather


def _sc_scatter(x2d, pdest_flat, prob_flat):
    return _sc_kernels()[0](x2d, pdest_flat, prob_flat)


def _sc_gather(out_pad, pdest_flat):
    return _sc_kernels()[1](out_pad, pdest_flat)


def _ffn_body(eot_ref, x_ref, w1_ref, b1_ref, w2_ref, b2_ref, o_ref):
    del eot_ref
    xx = x_ref[:, :D]
    pcol = x_ref[:, D:D + 1]
    h = jnp.dot(xx, w1_ref[0], preferred_element_type=jnp.float32)
    h = jnp.maximum(h + b1_ref[0], 0.0)
    o = jnp.dot(h, w2_ref[0], preferred_element_type=jnp.float32)
    o_ref[...] = (o + b2_ref[0]) * pcol


def _ffn(eot, x_pad, W1, b1, W2, b2):
    grid_spec = pltpu.PrefetchScalarGridSpec(
        num_scalar_prefetch=1,
        grid=(NT,),
        in_specs=[
            pl.BlockSpec((T, DP), lambda i, eot: (i, 0)),
            pl.BlockSpec((1, D, H), lambda i, eot: (eot[i], 0, 0)),
            pl.BlockSpec((1, 1, H), lambda i, eot: (eot[i], 0, 0)),
            pl.BlockSpec((1, H, D), lambda i, eot: (eot[i], 0, 0)),
            pl.BlockSpec((1, 1, D), lambda i, eot: (eot[i], 0, 0)),
        ],
        out_specs=pl.BlockSpec((T, D), lambda i, eot: (i, 0)),
    )
    return pl.pallas_call(
        _ffn_body,
        grid_spec=grid_spec,
        out_shape=jax.ShapeDtypeStruct((NPAD, D), jnp.float32),
        compiler_params=pltpu.CompilerParams(
            dimension_semantics=("arbitrary",)),
    )(eot, x_pad, W1, b1, W2, b2)


def kernel(x, router_W, router_b, W1, b1, W2, b2):
    x2d = x.reshape(N, D)
    wt = router_W.T
    rb = router_b.reshape(1, E)

    eid, prob, counts, bloss = _router(x2d, wt, rb)
    pdest, eot = _meta(counts, eid)
    pdest_flat = pdest.reshape(N)
    eot_flat = eot.reshape(128)[:NT]

    x_pad = _sc_scatter(x2d, pdest_flat, prob.reshape(N))
    out_pad = _ffn(eot_flat, x_pad, W1, b1.reshape(E, 1, H), W2,
                   b2.reshape(E, 1, D))
    out = _sc_gather(out_pad, pdest_flat)

    return out.reshape(x.shape), bloss.reshape(())


# speedup vs baseline: 1.0544x; 1.0544x over previous
import functools

import jax
import jax.numpy as jnp
from jax import lax
from jax.experimental import pallas as pl
from jax.experimental.pallas import tpu as pltpu
from jax.experimental.pallas import tpu_sc as plsc

E = 8
D = 768
H = 768
N = 8192
T = 256
NT = N // T + E
NPAD = NT * T
RT = 1024
RG = N // RT

NW = 32
CHUNK = 64
TPW = N // NW
CPW = TPW // CHUNK
PX = D // 2
PW = 512


def _pack_bf16(x):
    u = pltpu.bitcast(x, jnp.uint32)
    rnd = (u + 0x7FFF + ((u >> 16) & 1)) >> 16
    return rnd[:, :PX] | (rnd[:, PX:] << 16)


def _unpack_bf16(p):
    lo = pltpu.bitcast(p << 16, jnp.float32)
    hi = pltpu.bitcast(p & jnp.uint32(0xFFFF0000), jnp.float32)
    return jnp.concatenate([lo, hi], axis=1).astype(jnp.bfloat16)


def _router_body(x_ref, wt_ref, b_ref, eid_ref, prob_ref, counts_ref,
                 bloss_ref, xbf_ref, cnt_s, ps_s):
    i = pl.program_id(0)

    @pl.when(i == 0)
    def _init():
        cnt_s[...] = jnp.zeros_like(cnt_s)
        ps_s[...] = jnp.zeros_like(ps_s)

    x = x_ref[...]
    logits = jnp.dot(x, wt_ref[...], preferred_element_type=jnp.float32)
    logits = logits + b_ref[...]
    m = jnp.max(logits, axis=1, keepdims=True)
    ex = jnp.exp(logits - m)
    s = jnp.sum(ex, axis=1, keepdims=True)
    probs = ex / s
    pmax = jnp.max(probs, axis=1, keepdims=True)
    iota_e = lax.broadcasted_iota(jnp.int32, (RT, E), 1)
    eid = jnp.min(jnp.where(probs == pmax, iota_e, E), axis=1, keepdims=True)
    eid_ref[...] = eid
    prob_ref[...] = pmax
    xbf_ref[...] = jnp.concatenate(
        [_pack_bf16(x), pltpu.bitcast(pmax, jnp.uint32),
         jnp.zeros((RT, PW - PX - 1), jnp.uint32)], axis=1)
    oh = (eid == iota_e).astype(jnp.float32)
    cnt_s[...] += jnp.sum(oh, axis=0, keepdims=True)
    ps_s[...] += jnp.sum(probs, axis=0, keepdims=True)
    counts_ref[...] = cnt_s[...]
    bloss_ref[...] = (E / (N * N)) * jnp.sum(
        cnt_s[...] * ps_s[...], axis=1, keepdims=True)


def _router(x2d, wt, rb):
    return pl.pallas_call(
        _router_body,
        grid=(RG,),
        in_specs=[
            pl.BlockSpec((RT, D), lambda i: (i, 0)),
            pl.BlockSpec((D, E), lambda i: (0, 0)),
            pl.BlockSpec((1, E), lambda i: (0, 0)),
        ],
        out_specs=[
            pl.BlockSpec((RT, 1), lambda i: (i, 0)),
            pl.BlockSpec((RT, 1), lambda i: (i, 0)),
            pl.BlockSpec((1, E), lambda i: (0, 0)),
            pl.BlockSpec((1, 1), lambda i: (0, 0)),
            pl.BlockSpec((RT, PW), lambda i: (i, 0)),
        ],
        out_shape=[
            jax.ShapeDtypeStruct((N, 1), jnp.int32),
            jax.ShapeDtypeStruct((N, 1), jnp.float32),
            jax.ShapeDtypeStruct((1, E), jnp.float32),
            jax.ShapeDtypeStruct((1, 1), jnp.float32),
            jax.ShapeDtypeStruct((N, PW), jnp.uint32),
        ],
        scratch_shapes=[
            pltpu.VMEM((1, E), jnp.float32),
            pltpu.VMEM((1, E), jnp.float32),
        ],
        compiler_params=pltpu.CompilerParams(
            dimension_semantics=("arbitrary",)),
    )(x2d, wt, rb)


def _meta_body(counts_ref, eid_ref, pdest_ref, eot_ref, run_s):
    i = pl.program_id(0)

    @pl.when(i == 0)
    def _init():
        run_s[...] = jnp.zeros_like(run_s)

    poff = []
    off = 0.0
    for e in range(E):
        c = counts_ref[0, e]
        pc = jnp.ceil(c / T) * T
        poff.append(off)
        off = off + pc

    iota_lane = lax.broadcasted_iota(jnp.int32, (1, E), 1)
    base = jnp.zeros((1, E), jnp.float32)
    for e in range(E):
        base = base + jnp.where(iota_lane == e, poff[e], 0.0)
    base = base + run_s[...]

    eid = eid_ref[...]
    iota_e = lax.broadcasted_iota(jnp.int32, (RT, E), 1)
    oh = (eid == iota_e).astype(jnp.float32)
    csum = oh
    k = 1
    while k < RT:
        csum = csum + jnp.concatenate(
            [jnp.zeros((k, E), jnp.float32), csum[:-k, :]], axis=0)
        k *= 2
    pd = jnp.sum(oh * (base + csum - 1.0), axis=1, keepdims=True)
    pdest_ref[...] = pd.astype(jnp.int32)
    run_s[...] += jnp.sum(oh, axis=0, keepdims=True)

    tgrid = lax.broadcasted_iota(jnp.int32, (1, 128), 1).astype(jnp.float32) * T
    acc = jnp.zeros((1, 128), jnp.float32)
    for e in range(E):
        acc = acc + (poff[e] <= tgrid).astype(jnp.float32)
    eot_ref[...] = (acc - 1.0).astype(jnp.int32)


def _meta(counts, eid):
    return pl.pallas_call(
        _meta_body,
        grid=(RG,),
        in_specs=[
            pl.BlockSpec(memory_space=pltpu.SMEM),
            pl.BlockSpec((RT, 1), lambda i: (i, 0)),
        ],
        out_specs=[
            pl.BlockSpec((RT, 1), lambda i: (i, 0)),
            pl.BlockSpec((1, 128), lambda i: (0, 0)),
        ],
        out_shape=[
            jax.ShapeDtypeStruct((N, 1), jnp.int32),
            jax.ShapeDtypeStruct((1, 128), jnp.int32),
        ],
        scratch_shapes=[pltpu.VMEM((1, E), jnp.float32)],
        compiler_params=pltpu.CompilerParams(
            dimension_semantics=("arbitrary",)),
    )(counts, eid)


@functools.cache
def _sc_kernels():
    mesh = plsc.VectorSubcoreMesh(core_axis_name="c", subcore_axis_name="s")

    @functools.partial(
        pl.kernel,
        out_type=jax.ShapeDtypeStruct((NPAD, PW), jnp.uint32),
        mesh=mesh,
        scratch_types=[
            pltpu.VMEM((CPW, CHUNK), jnp.int32),
            pltpu.VMEM((2, CHUNK, PW), jnp.uint32),
            pltpu.SemaphoreType.DMA,
            pltpu.SemaphoreType.DMA,
            pltpu.SemaphoreType.DMA,
        ],
        compiler_params=pltpu.CompilerParams(needs_layout_passes=False))
    def sc_scatter(x_hbm, pdest_hbm, xpad_hbm,
                   idx_v, rows_v, sem_i, sem_s0, sem_s1):
        wid = lax.axis_index("s") * 2 + lax.axis_index("c")
        wbase = wid * TPW
        sems = [sem_s0, sem_s1]
        loads = [pltpu.make_async_copy(
            pdest_hbm.at[pl.ds(wbase + ch * CHUNK, CHUNK)], idx_v.at[ch],
            sem_i) for ch in range(CPW)]
        for cp in loads:
            cp.start()
        for cp in loads:
            cp.wait()
        scatters = [None, None]
        for ch in range(CPW):
            slot = ch % 2
            if scatters[slot] is not None:
                scatters[slot].wait()
            pltpu.sync_copy(x_hbm.at[pl.ds(wbase + ch * CHUNK, CHUNK)],
                            rows_v.at[slot])
            scatters[slot] = pltpu.make_async_copy(
                rows_v.at[slot], xpad_hbm.at[idx_v.at[ch]], sems[slot])
            scatters[slot].start()
        for cp in scatters:
            cp.wait()

    @functools.partial(
        pl.kernel,
        out_type=jax.ShapeDtypeStruct((N, D), jnp.float32),
        mesh=mesh,
        scratch_types=[
            pltpu.VMEM((TPW,), jnp.int32),
            pltpu.VMEM((2, CHUNK, D), jnp.float32),
            pltpu.SemaphoreType.DMA,
            pltpu.SemaphoreType.DMA,
        ])
    def sc_gather(opad_hbm, pdest_hbm, out_hbm, idx_v, rows_v, sem_g0, sem_g1):
        wid = lax.axis_index("s") * 2 + lax.axis_index("c")
        wbase = wid * TPW
        sems = [sem_g0, sem_g1]
        pltpu.sync_copy(pdest_hbm.at[pl.ds(wbase, TPW)], idx_v)

        def gather(ch):
            cp = pltpu.make_async_copy(
                opad_hbm.at[idx_v.at[pl.ds(ch * CHUNK, CHUNK)]],
                rows_v.at[ch % 2], sems[ch % 2])
            cp.start()
            return cp

        pending = gather(0)
        for ch in range(CPW):
            nxt = gather(ch + 1) if ch + 1 < CPW else None
            pending.wait()
            pltpu.sync_copy(rows_v.at[ch % 2],
                            out_hbm.at[pl.ds(wbase + ch * CHUNK, CHUNK)])
            pending = nxt

    return sc_scatter, sc_gather


def _sc_scatter(x_packed, pdest_flat):
    return _sc_kernels()[0](x_packed, pdest_flat)


def _sc_gather(out_pad, pdest_flat):
    return _sc_kernels()[1](out_pad, pdest_flat)


def _ffn_body(eot_ref, x_ref, w1_ref, b1_ref, w2_ref, b2_ref, o_ref,
              w1_s, w2_s):
    i = pl.program_id(0)
    prev = eot_ref[jnp.maximum(i - 1, 0)]
    recast = jnp.logical_or(i == 0, prev != eot_ref[i])

    @pl.when(recast)
    def _recast():
        w1_s[...] = w1_ref[0].astype(jnp.bfloat16)
        w2_s[...] = w2_ref[0].astype(jnp.bfloat16)

    h = jnp.dot(_unpack_bf16(x_ref[:, :PX]), w1_s[...],
                preferred_element_type=jnp.float32)
    h = jnp.maximum(h + b1_ref[0], 0.0).astype(jnp.bfloat16)
    o = jnp.dot(h, w2_s[...], preferred_element_type=jnp.float32)
    pcol = pltpu.bitcast(x_ref[:, PX:PX + 1], jnp.float32)
    o_ref[...] = (o + b2_ref[0]) * pcol


def _ffn(eot, x_pad, W1, b1, W2, b2):
    grid_spec = pltpu.PrefetchScalarGridSpec(
        num_scalar_prefetch=1,
        grid=(NT,),
        in_specs=[
            pl.BlockSpec((T, PW), lambda i, eot: (i, 0)),
            pl.BlockSpec((1, D, H), lambda i, eot: (eot[i], 0, 0)),
            pl.BlockSpec((1, 1, H), lambda i, eot: (eot[i], 0, 0)),
            pl.BlockSpec((1, H, D), lambda i, eot: (eot[i], 0, 0)),
            pl.BlockSpec((1, 1, D), lambda i, eot: (eot[i], 0, 0)),
        ],
        out_specs=pl.BlockSpec((T, D), lambda i, eot: (i, 0)),
        scratch_shapes=[
            pltpu.VMEM((D, H), jnp.bfloat16),
            pltpu.VMEM((H, D), jnp.bfloat16),
        ],
    )
    return pl.pallas_call(
        _ffn_body,
        grid_spec=grid_spec,
        out_shape=jax.ShapeDtypeStruct((NPAD, D), jnp.float32),
        compiler_params=pltpu.CompilerParams(
            dimension_semantics=("arbitrary",)),
    )(eot, x_pad, W1, b1, W2, b2)


def kernel(x, router_W, router_b, W1, b1, W2, b2):
    x2d = x.reshape(N, D)
    wt = router_W.T
    rb = router_b.reshape(1, E)

    eid, prob, counts, bloss, x_bf = _router(x2d, wt, rb)
    pdest, eot = _meta(counts, eid)
    pdest_flat = pdest.reshape(N)
    eot_flat = eot.reshape(128)[:NT]

    x_pad = _sc_scatter(x_bf, pdest_flat)
    out_pad = _ffn(eot_flat, x_pad, W1, b1.reshape(E, 1, H), W2,
                   b2.reshape(E, 1, D))
    out = _sc_gather(out_pad, pdest_flat)

    return out.reshape(x.shape), bloss.reshape(())


# speedup vs baseline: 1.0775x; 1.0219x over previous
import functools

import jax
import jax.numpy as jnp
from jax import lax
from jax.experimental import pallas as pl
from jax.experimental.pallas import tpu as pltpu
from jax.experimental.pallas import tpu_sc as plsc

E = 8
D = 768
H = 768
N = 8192
T = 256
NT = N // T + E
NPAD = NT * T
RT = 2048
RG = N // RT

NW = 32
CHUNK = 64
TPW = N // NW
CPW = TPW // CHUNK
PX = D // 2
PW = 512


def _pack_bf16(x):
    u = pltpu.bitcast(x, jnp.uint32)
    rnd = (u + 0x7FFF + ((u >> 16) & 1)) >> 16
    return rnd[:, :PX] | (rnd[:, PX:] << 16)


def _unpack_bf16(p):
    lo = pltpu.bitcast(p << 16, jnp.float32)
    hi = pltpu.bitcast(p & jnp.uint32(0xFFFF0000), jnp.float32)
    return jnp.concatenate([lo, hi], axis=1).astype(jnp.bfloat16)


def _router_body(x_ref, wt_ref, b_ref, eid_ref, prob_ref, counts_ref,
                 bloss_ref, xbf_ref, cnt_s, ps_s):
    i = pl.program_id(0)

    @pl.when(i == 0)
    def _init():
        cnt_s[...] = jnp.zeros_like(cnt_s)
        ps_s[...] = jnp.zeros_like(ps_s)

    x = x_ref[...]
    logits = jnp.dot(x, wt_ref[...], preferred_element_type=jnp.float32)
    logits = logits + b_ref[...]
    m = jnp.max(logits, axis=1, keepdims=True)
    ex = jnp.exp(logits - m)
    s = jnp.sum(ex, axis=1, keepdims=True)
    probs = ex / s
    pmax = jnp.max(probs, axis=1, keepdims=True)
    iota_e = lax.broadcasted_iota(jnp.int32, (RT, E), 1)
    eid = jnp.min(jnp.where(probs == pmax, iota_e, E), axis=1, keepdims=True)
    eid_ref[...] = eid
    prob_ref[...] = pmax
    xbf_ref[...] = jnp.concatenate(
        [_pack_bf16(x), pltpu.bitcast(pmax, jnp.uint32),
         jnp.zeros((RT, PW - PX - 1), jnp.uint32)], axis=1)
    oh = (eid == iota_e).astype(jnp.float32)
    cnt_s[...] += jnp.sum(oh, axis=0, keepdims=True)
    ps_s[...] += jnp.sum(probs, axis=0, keepdims=True)
    counts_ref[...] = cnt_s[...]
    bloss_ref[...] = (E / (N * N)) * jnp.sum(
        cnt_s[...] * ps_s[...], axis=1, keepdims=True)


def _router(x2d, wt, rb):
    return pl.pallas_call(
        _router_body,
        grid=(RG,),
        in_specs=[
            pl.BlockSpec((RT, D), lambda i: (i, 0)),
            pl.BlockSpec((D, E), lambda i: (0, 0)),
            pl.BlockSpec((1, E), lambda i: (0, 0)),
        ],
        out_specs=[
            pl.BlockSpec((RT, 1), lambda i: (i, 0)),
            pl.BlockSpec((RT, 1), lambda i: (i, 0)),
            pl.BlockSpec((1, E), lambda i: (0, 0)),
            pl.BlockSpec((1, 1), lambda i: (0, 0)),
            pl.BlockSpec((RT, PW), lambda i: (i, 0)),
        ],
        out_shape=[
            jax.ShapeDtypeStruct((N, 1), jnp.int32),
            jax.ShapeDtypeStruct((N, 1), jnp.float32),
            jax.ShapeDtypeStruct((1, E), jnp.float32),
            jax.ShapeDtypeStruct((1, 1), jnp.float32),
            jax.ShapeDtypeStruct((N, PW), jnp.uint32),
        ],
        scratch_shapes=[
            pltpu.VMEM((1, E), jnp.float32),
            pltpu.VMEM((1, E), jnp.float32),
        ],
        compiler_params=pltpu.CompilerParams(
            dimension_semantics=("arbitrary",)),
    )(x2d, wt, rb)


def _meta_body(counts_ref, eid_ref, pdest_ref, eot_ref, run_s):
    i = pl.program_id(0)

    @pl.when(i == 0)
    def _init():
        run_s[...] = jnp.zeros_like(run_s)

    poff = []
    off = 0.0
    for e in range(E):
        c = counts_ref[0, e]
        pc = jnp.ceil(c / T) * T
        poff.append(off)
        off = off + pc

    iota_lane = lax.broadcasted_iota(jnp.int32, (1, E), 1)
    base = jnp.zeros((1, E), jnp.float32)
    for e in range(E):
        base = base + jnp.where(iota_lane == e, poff[e], 0.0)
    base = base + run_s[...]

    eid = eid_ref[...]
    iota_e = lax.broadcasted_iota(jnp.int32, (RT, E), 1)
    oh = (eid == iota_e).astype(jnp.float32)
    csum = oh
    k = 1
    while k < RT:
        csum = csum + jnp.concatenate(
            [jnp.zeros((k, E), jnp.float32), csum[:-k, :]], axis=0)
        k *= 2
    pd = jnp.sum(oh * (base + csum - 1.0), axis=1, keepdims=True)
    pdest_ref[...] = pd.astype(jnp.int32)
    run_s[...] += jnp.sum(oh, axis=0, keepdims=True)

    tgrid = lax.broadcasted_iota(jnp.int32, (1, 128), 1).astype(jnp.float32) * T
    acc = jnp.zeros((1, 128), jnp.float32)
    for e in range(E):
        acc = acc + (poff[e] <= tgrid).astype(jnp.float32)
    eot_ref[...] = (acc - 1.0).astype(jnp.int32)


def _meta(counts, eid):
    return pl.pallas_call(
        _meta_body,
        grid=(RG,),
        in_specs=[
            pl.BlockSpec(memory_space=pltpu.SMEM),
            pl.BlockSpec((RT, 1), lambda i: (i, 0)),
        ],
        out_specs=[
            pl.BlockSpec((RT, 1), lambda i: (i, 0)),
            pl.BlockSpec((1, 128), lambda i: (0, 0)),
        ],
        out_shape=[
            jax.ShapeDtypeStruct((N, 1), jnp.int32),
            jax.ShapeDtypeStruct((1, 128), jnp.int32),
        ],
        scratch_shapes=[pltpu.VMEM((1, E), jnp.float32)],
        compiler_params=pltpu.CompilerParams(
            dimension_semantics=("arbitrary",)),
    )(counts, eid)


@functools.cache
def _sc_kernels():
    mesh = plsc.VectorSubcoreMesh(core_axis_name="c", subcore_axis_name="s")

    @functools.partial(
        pl.kernel,
        out_type=jax.ShapeDtypeStruct((NPAD, PW), jnp.uint32),
        mesh=mesh,
        scratch_types=[
            pltpu.VMEM((CPW, CHUNK), jnp.int32),
            pltpu.VMEM((2, CHUNK, PW), jnp.uint32),
            pltpu.SemaphoreType.DMA,
            pltpu.SemaphoreType.DMA,
            pltpu.SemaphoreType.DMA,
        ],
        compiler_params=pltpu.CompilerParams(needs_layout_passes=False))
    def sc_scatter(x_hbm, pdest_hbm, xpad_hbm,
                   idx_v, rows_v, sem_i, sem_s0, sem_s1):
        wid = lax.axis_index("s") * 2 + lax.axis_index("c")
        wbase = wid * TPW
        sems = [sem_s0, sem_s1]
        loads = [pltpu.make_async_copy(
            pdest_hbm.at[pl.ds(wbase + ch * CHUNK, CHUNK)], idx_v.at[ch],
            sem_i) for ch in range(CPW)]
        for cp in loads:
            cp.start()
        for cp in loads:
            cp.wait()
        scatters = [None, None]
        for ch in range(CPW):
            slot = ch % 2
            if scatters[slot] is not None:
                scatters[slot].wait()
            pltpu.sync_copy(x_hbm.at[pl.ds(wbase + ch * CHUNK, CHUNK)],
                            rows_v.at[slot])
            scatters[slot] = pltpu.make_async_copy(
                rows_v.at[slot], xpad_hbm.at[idx_v.at[ch]], sems[slot])
            scatters[slot].start()
        for cp in scatters:
            cp.wait()

    @functools.partial(
        pl.kernel,
        out_type=jax.ShapeDtypeStruct((N, D), jnp.float32),
        mesh=mesh,
        scratch_types=[
            pltpu.VMEM((TPW,), jnp.int32),
            pltpu.VMEM((2, CHUNK, D), jnp.float32),
            pltpu.SemaphoreType.DMA,
            pltpu.SemaphoreType.DMA,
        ])
    def sc_gather(opad_hbm, pdest_hbm, out_hbm, idx_v, rows_v, sem_g0, sem_g1):
        wid = lax.axis_index("s") * 2 + lax.axis_index("c")
        wbase = wid * TPW
        sems = [sem_g0, sem_g1]
        pltpu.sync_copy(pdest_hbm.at[pl.ds(wbase, TPW)], idx_v)

        def gather(ch):
            cp = pltpu.make_async_copy(
                opad_hbm.at[idx_v.at[pl.ds(ch * CHUNK, CHUNK)]],
                rows_v.at[ch % 2], sems[ch % 2])
            cp.start()
            return cp

        pending = gather(0)
        for ch in range(CPW):
            nxt = gather(ch + 1) if ch + 1 < CPW else None
            pending.wait()
            pltpu.sync_copy(rows_v.at[ch % 2],
                            out_hbm.at[pl.ds(wbase + ch * CHUNK, CHUNK)])
            pending = nxt

    return sc_scatter, sc_gather


def _sc_scatter(x_packed, pdest_flat):
    return _sc_kernels()[0](x_packed, pdest_flat)


def _sc_gather(out_pad, pdest_flat):
    return _sc_kernels()[1](out_pad, pdest_flat)


def _ffn_body(eot_ref, x_ref, w1_ref, b1_ref, w2_ref, b2_ref, o_ref,
              w1_s, w2_s):
    i = pl.program_id(0)
    prev = eot_ref[jnp.maximum(i - 1, 0)]
    recast = jnp.logical_or(i == 0, prev != eot_ref[i])

    @pl.when(recast)
    def _recast():
        w1_s[...] = w1_ref[0].astype(jnp.bfloat16)
        w2_s[...] = w2_ref[0].astype(jnp.bfloat16)

    h = jnp.dot(_unpack_bf16(x_ref[:, :PX]), w1_s[...],
                preferred_element_type=jnp.float32)
    h = jnp.maximum(h + b1_ref[0], 0.0).astype(jnp.bfloat16)
    o = jnp.dot(h, w2_s[...], preferred_element_type=jnp.float32)
    pcol = pltpu.bitcast(x_ref[:, PX:PX + 1], jnp.float32)
    o_ref[...] = (o + b2_ref[0]) * pcol


def _ffn(eot, x_pad, W1, b1, W2, b2):
    grid_spec = pltpu.PrefetchScalarGridSpec(
        num_scalar_prefetch=1,
        grid=(NT,),
        in_specs=[
            pl.BlockSpec((T, PW), lambda i, eot: (i, 0)),
            pl.BlockSpec((1, D, H), lambda i, eot: (eot[i], 0, 0)),
            pl.BlockSpec((1, 1, H), lambda i, eot: (eot[i], 0, 0)),
            pl.BlockSpec((1, H, D), lambda i, eot: (eot[i], 0, 0)),
            pl.BlockSpec((1, 1, D), lambda i, eot: (eot[i], 0, 0)),
        ],
        out_specs=pl.BlockSpec((T, D), lambda i, eot: (i, 0)),
        scratch_shapes=[
            pltpu.VMEM((D, H), jnp.bfloat16),
            pltpu.VMEM((H, D), jnp.bfloat16),
        ],
    )
    return pl.pallas_call(
        _ffn_body,
        grid_spec=grid_spec,
        out_shape=jax.ShapeDtypeStruct((NPAD, D), jnp.float32),
        compiler_params=pltpu.CompilerParams(
            dimension_semantics=("arbitrary",)),
    )(eot, x_pad, W1, b1, W2, b2)


def kernel(x, router_W, router_b, W1, b1, W2, b2):
    x2d = x.reshape(N, D)
    wt = router_W.T
    rb = router_b.reshape(1, E)

    eid, prob, counts, bloss, x_bf = _router(x2d, wt, rb)
    pdest, eot = _meta(counts, eid)
    pdest_flat = pdest.reshape(N)
    eot_flat = eot.reshape(128)[:NT]

    x_pad = _sc_scatter(x_bf, pdest_flat)
    out_pad = _ffn(eot_flat, x_pad, W1, b1.reshape(E, 1, H), W2,
                   b2.reshape(E, 1, D))
    out = _sc_gather(out_pad, pdest_flat)

    return out.reshape(x.shape), bloss.reshape(())


# speedup vs baseline: 1.0796x; 1.0020x over previous
import functools

import jax
import jax.numpy as jnp
from jax import lax
from jax.experimental import pallas as pl
from jax.experimental.pallas import tpu as pltpu
from jax.experimental.pallas import tpu_sc as plsc

E = 8
D = 768
H = 768
N = 8192
T = 256
NT = N // T + E
NPAD = NT * T
RT = 4096
RG = N // RT

NW = 32
CHUNK = 64
TPW = N // NW
CPW = TPW // CHUNK
PX = D // 2
PW = 512


def _pack_bf16(x):
    u = pltpu.bitcast(x, jnp.uint32)
    rnd = (u + 0x7FFF + ((u >> 16) & 1)) >> 16
    return rnd[:, :PX] | (rnd[:, PX:] << 16)


def _unpack_bf16(p):
    lo = pltpu.bitcast(p << 16, jnp.float32)
    hi = pltpu.bitcast(p & jnp.uint32(0xFFFF0000), jnp.float32)
    return jnp.concatenate([lo, hi], axis=1).astype(jnp.bfloat16)


def _router_body(x_ref, wt_ref, b_ref, eid_ref, prob_ref, counts_ref,
                 bloss_ref, xbf_ref, cnt_s, ps_s):
    i = pl.program_id(0)

    @pl.when(i == 0)
    def _init():
        cnt_s[...] = jnp.zeros_like(cnt_s)
        ps_s[...] = jnp.zeros_like(ps_s)

    x = x_ref[...]
    logits = jnp.dot(x, wt_ref[...], preferred_element_type=jnp.float32)
    logits = logits + b_ref[...]
    m = jnp.max(logits, axis=1, keepdims=True)
    ex = jnp.exp(logits - m)
    s = jnp.sum(ex, axis=1, keepdims=True)
    probs = ex / s
    pmax = jnp.max(probs, axis=1, keepdims=True)
    iota_e = lax.broadcasted_iota(jnp.int32, (RT, E), 1)
    eid = jnp.min(jnp.where(probs == pmax, iota_e, E), axis=1, keepdims=True)
    eid_ref[...] = eid
    prob_ref[...] = pmax
    xbf_ref[...] = jnp.concatenate(
        [_pack_bf16(x), pltpu.bitcast(pmax, jnp.uint32),
         jnp.zeros((RT, PW - PX - 1), jnp.uint32)], axis=1)
    oh = (eid == iota_e).astype(jnp.float32)
    cnt_s[...] += jnp.sum(oh, axis=0, keepdims=True)
    ps_s[...] += jnp.sum(probs, axis=0, keepdims=True)
    counts_ref[...] = cnt_s[...]
    bloss_ref[...] = (E / (N * N)) * jnp.sum(
        cnt_s[...] * ps_s[...], axis=1, keepdims=True)


def _router(x2d, wt, rb):
    return pl.pallas_call(
        _router_body,
        grid=(RG,),
        in_specs=[
            pl.BlockSpec((RT, D), lambda i: (i, 0)),
            pl.BlockSpec((D, E), lambda i: (0, 0)),
            pl.BlockSpec((1, E), lambda i: (0, 0)),
        ],
        out_specs=[
            pl.BlockSpec((RT, 1), lambda i: (i, 0)),
            pl.BlockSpec((RT, 1), lambda i: (i, 0)),
            pl.BlockSpec((1, E), lambda i: (0, 0)),
            pl.BlockSpec((1, 1), lambda i: (0, 0)),
            pl.BlockSpec((RT, PW), lambda i: (i, 0)),
        ],
        out_shape=[
            jax.ShapeDtypeStruct((N, 1), jnp.int32),
            jax.ShapeDtypeStruct((N, 1), jnp.float32),
            jax.ShapeDtypeStruct((1, E), jnp.float32),
            jax.ShapeDtypeStruct((1, 1), jnp.float32),
            jax.ShapeDtypeStruct((N, PW), jnp.uint32),
        ],
        scratch_shapes=[
            pltpu.VMEM((1, E), jnp.float32),
            pltpu.VMEM((1, E), jnp.float32),
        ],
        compiler_params=pltpu.CompilerParams(
            dimension_semantics=("arbitrary",)),
    )(x2d, wt, rb)


def _meta_body(counts_ref, eid_ref, pdest_ref, eot_ref, run_s):
    i = pl.program_id(0)

    @pl.when(i == 0)
    def _init():
        run_s[...] = jnp.zeros_like(run_s)

    poff = []
    off = 0.0
    for e in range(E):
        c = counts_ref[0, e]
        pc = jnp.ceil(c / T) * T
        poff.append(off)
        off = off + pc

    iota_lane = lax.broadcasted_iota(jnp.int32, (1, E), 1)
    base = jnp.zeros((1, E), jnp.float32)
    for e in range(E):
        base = base + jnp.where(iota_lane == e, poff[e], 0.0)
    base = base + run_s[...]

    eid = eid_ref[...]
    iota_e = lax.broadcasted_iota(jnp.int32, (RT, E), 1)
    oh = (eid == iota_e).astype(jnp.float32)
    csum = oh
    k = 1
    while k < RT:
        csum = csum + jnp.concatenate(
            [jnp.zeros((k, E), jnp.float32), csum[:-k, :]], axis=0)
        k *= 2
    pd = jnp.sum(oh * (base + csum - 1.0), axis=1, keepdims=True)
    pdest_ref[...] = pd.astype(jnp.int32)
    run_s[...] += jnp.sum(oh, axis=0, keepdims=True)

    tgrid = lax.broadcasted_iota(jnp.int32, (1, 128), 1).astype(jnp.float32) * T
    acc = jnp.zeros((1, 128), jnp.float32)
    for e in range(E):
        acc = acc + (poff[e] <= tgrid).astype(jnp.float32)
    eot_ref[...] = (acc - 1.0).astype(jnp.int32)


def _meta(counts, eid):
    return pl.pallas_call(
        _meta_body,
        grid=(RG,),
        in_specs=[
            pl.BlockSpec(memory_space=pltpu.SMEM),
            pl.BlockSpec((RT, 1), lambda i: (i, 0)),
        ],
        out_specs=[
            pl.BlockSpec((RT, 1), lambda i: (i, 0)),
            pl.BlockSpec((1, 128), lambda i: (0, 0)),
        ],
        out_shape=[
            jax.ShapeDtypeStruct((N, 1), jnp.int32),
            jax.ShapeDtypeStruct((1, 128), jnp.int32),
        ],
        scratch_shapes=[pltpu.VMEM((1, E), jnp.float32)],
        compiler_params=pltpu.CompilerParams(
            dimension_semantics=("arbitrary",)),
    )(counts, eid)


@functools.cache
def _sc_kernels():
    mesh = plsc.VectorSubcoreMesh(core_axis_name="c", subcore_axis_name="s")

    @functools.partial(
        pl.kernel,
        out_type=jax.ShapeDtypeStruct((NPAD, PW), jnp.uint32),
        mesh=mesh,
        scratch_types=[
            pltpu.VMEM((CPW, CHUNK), jnp.int32),
            pltpu.VMEM((2, CHUNK, PW), jnp.uint32),
            pltpu.SemaphoreType.DMA,
            pltpu.SemaphoreType.DMA,
            pltpu.SemaphoreType.DMA,
        ],
        compiler_params=pltpu.CompilerParams(needs_layout_passes=False))
    def sc_scatter(x_hbm, pdest_hbm, xpad_hbm,
                   idx_v, rows_v, sem_i, sem_s0, sem_s1):
        wid = lax.axis_index("s") * 2 + lax.axis_index("c")
        wbase = wid * TPW
        sems = [sem_s0, sem_s1]
        loads = [pltpu.make_async_copy(
            pdest_hbm.at[pl.ds(wbase + ch * CHUNK, CHUNK)], idx_v.at[ch],
            sem_i) for ch in range(CPW)]
        for cp in loads:
            cp.start()
        for cp in loads:
            cp.wait()
        scatters = [None, None]
        for ch in range(CPW):
            slot = ch % 2
            if scatters[slot] is not None:
                scatters[slot].wait()
            pltpu.sync_copy(x_hbm.at[pl.ds(wbase + ch * CHUNK, CHUNK)],
                            rows_v.at[slot])
            scatters[slot] = pltpu.make_async_copy(
                rows_v.at[slot], xpad_hbm.at[idx_v.at[ch]], sems[slot])
            scatters[slot].start()
        for cp in scatters:
            cp.wait()

    @functools.partial(
        pl.kernel,
        out_type=jax.ShapeDtypeStruct((N, D), jnp.float32),
        mesh=mesh,
        scratch_types=[
            pltpu.VMEM((TPW,), jnp.int32),
            pltpu.VMEM((2, CHUNK, D), jnp.float32),
            pltpu.SemaphoreType.DMA,
            pltpu.SemaphoreType.DMA,
        ])
    def sc_gather(opad_hbm, pdest_hbm, out_hbm, idx_v, rows_v, sem_g0, sem_g1):
        wid = lax.axis_index("s") * 2 + lax.axis_index("c")
        wbase = wid * TPW
        sems = [sem_g0, sem_g1]
        pltpu.sync_copy(pdest_hbm.at[pl.ds(wbase, TPW)], idx_v)

        def gather(ch):
            cp = pltpu.make_async_copy(
                opad_hbm.at[idx_v.at[pl.ds(ch * CHUNK, CHUNK)]],
                rows_v.at[ch % 2], sems[ch % 2])
            cp.start()
            return cp

        pending = gather(0)
        for ch in range(CPW):
            nxt = gather(ch + 1) if ch + 1 < CPW else None
            pending.wait()
            pltpu.sync_copy(rows_v.at[ch % 2],
                            out_hbm.at[pl.ds(wbase + ch * CHUNK, CHUNK)])
            pending = nxt

    return sc_scatter, sc_gather


def _sc_scatter(x_packed, pdest_flat):
    return _sc_kernels()[0](x_packed, pdest_flat)


def _sc_gather(out_pad, pdest_flat):
    return _sc_kernels()[1](out_pad, pdest_flat)


def _ffn_body(eot_ref, x_ref, w1_ref, b1_ref, w2_ref, b2_ref, o_ref,
              w1_s, w2_s):
    i = pl.program_id(0)
    prev = eot_ref[jnp.maximum(i - 1, 0)]
    recast = jnp.logical_or(i == 0, prev != eot_ref[i])

    @pl.when(recast)
    def _recast():
        w1_s[...] = w1_ref[0].astype(jnp.bfloat16)
        w2_s[...] = w2_ref[0].astype(jnp.bfloat16)

    h = jnp.dot(_unpack_bf16(x_ref[:, :PX]), w1_s[...],
                preferred_element_type=jnp.float32)
    h = jnp.maximum(h + b1_ref[0], 0.0).astype(jnp.bfloat16)
    o = jnp.dot(h, w2_s[...], preferred_element_type=jnp.float32)
    pcol = pltpu.bitcast(x_ref[:, PX:PX + 1], jnp.float32)
    o_ref[...] = (o + b2_ref[0]) * pcol


def _ffn(eot, x_pad, W1, b1, W2, b2):
    grid_spec = pltpu.PrefetchScalarGridSpec(
        num_scalar_prefetch=1,
        grid=(NT,),
        in_specs=[
            pl.BlockSpec((T, PW), lambda i, eot: (i, 0)),
            pl.BlockSpec((1, D, H), lambda i, eot: (eot[i], 0, 0)),
            pl.BlockSpec((1, 1, H), lambda i, eot: (eot[i], 0, 0)),
            pl.BlockSpec((1, H, D), lambda i, eot: (eot[i], 0, 0)),
            pl.BlockSpec((1, 1, D), lambda i, eot: (eot[i], 0, 0)),
        ],
        out_specs=pl.BlockSpec((T, D), lambda i, eot: (i, 0)),
        scratch_shapes=[
            pltpu.VMEM((D, H), jnp.bfloat16),
            pltpu.VMEM((H, D), jnp.bfloat16),
        ],
    )
    return pl.pallas_call(
        _ffn_body,
        grid_spec=grid_spec,
        out_shape=jax.ShapeDtypeStruct((NPAD, D), jnp.float32),
        compiler_params=pltpu.CompilerParams(
            dimension_semantics=("arbitrary",)),
    )(eot, x_pad, W1, b1, W2, b2)


def kernel(x, router_W, router_b, W1, b1, W2, b2):
    x2d = x.reshape(N, D)
    wt = router_W.T
    rb = router_b.reshape(1, E)

    eid, prob, counts, bloss, x_bf = _router(x2d, wt, rb)
    pdest, eot = _meta(counts, eid)
    pdest_flat = pdest.reshape(N)
    eot_flat = eot.reshape(128)[:NT]

    x_pad = _sc_scatter(x_bf, pdest_flat)
    out_pad = _ffn(eot_flat, x_pad, W1, b1.reshape(E, 1, H), W2,
                   b2.reshape(E, 1, D))
    out = _sc_gather(out_pad, pdest_flat)

    return out.reshape(x.shape), bloss.reshape(())
